# baseline (speedup 1.0000x reference)
import functools

import jax
import jax.numpy as jnp
from jax import lax
from jax.experimental import pallas as pl
from jax.experimental.pallas import tpu as pltpu
from jax.experimental.pallas import tpu_sc as plsc

_NUM_WORKERS = 32
_RANK = 64
_LANES = 16


def _make_sc_kernel(batch):
    bpw = batch // _NUM_WORKERS
    nch = max(1, bpw // 128)
    ch = bpw // nch

    mesh = plsc.VectorSubcoreMesh(core_axis_name="c", subcore_axis_name="s")

    @functools.partial(
        pl.kernel,
        out_type=jax.ShapeDtypeStruct((_NUM_WORKERS, nch, ch), jnp.float32),
        mesh=mesh,
        compiler_params=pltpu.CompilerParams(use_tc_tiling_on_sc=False),
        scratch_types=[
            pltpu.VMEM((nch, ch), jnp.int32),
            pltpu.VMEM((nch, ch), jnp.int32),
            pltpu.VMEM((nch, ch, _RANK), jnp.float32),
            pltpu.VMEM((nch, ch, _RANK), jnp.float32),
            pltpu.VMEM((nch, ch), jnp.float32),
            pltpu.SemaphoreType.DMA,
        ],
    )
    def sc_kernel(users_hbm, items_hbm, u_hbm, v_hbm, out_hbm,
                  uidx, iidx, urows, vrows, outv, sem):
        cid = lax.axis_index("c")
        sid = lax.axis_index("s")
        wid = sid * 2 + cid

        pltpu.sync_copy(users_hbm.at[wid], uidx)
        pltpu.sync_copy(items_hbm.at[wid], iidx)

        copies = []
        for j in range(nch):
            copies.append(pltpu.async_copy(u_hbm.at[uidx.at[j]], urows.at[j], sem))
            copies.append(pltpu.async_copy(v_hbm.at[iidx.at[j]], vrows.at[j], sem))
        for cp in copies:
            cp.wait()

        perms = [jnp.arange(_LANES, dtype=jnp.int32) ^ d for d in (8, 4, 2, 1)]
        onehot = [lax.iota(jnp.int32, _LANES) == i for i in range(_LANES)]

        def lane_sum(x):
            for p in perms:
                x = x + x.at[p].get(mode="promise_in_bounds")
            return x

        for j in range(nch):
            def body(g, _, j=j):
                res = jnp.zeros((_LANES,), jnp.float32)
                for i in range(_LANES):
                    r = g * _LANES + i
                    acc = (urows[j, r, pl.ds(0, _LANES)]
                           * vrows[j, r, pl.ds(0, _LANES)])
                    for c in range(1, _RANK // _LANES):
                        acc = acc + (urows[j, r, pl.ds(c * _LANES, _LANES)]
                                     * vrows[j, r, pl.ds(c * _LANES, _LANES)])
                    res = jnp.where(onehot[i], lane_sum(acc), res)
                outv[j, pl.ds(g * _LANES, _LANES)] = res
                return 0

            lax.fori_loop(0, ch // _LANES, body, 0)

        pltpu.sync_copy(outv, out_hbm.at[wid])

    return sc_kernel


def kernel(users, items, U, V):
    batch = users.shape[0]
    bpw = batch // _NUM_WORKERS
    nch = max(1, bpw // 128)
    ch = bpw // nch
    users3 = users.astype(jnp.int32).reshape(_NUM_WORKERS, nch, ch)
    items3 = items.astype(jnp.int32).reshape(_NUM_WORKERS, nch, ch)
    out = _make_sc_kernel(batch)(users3, items3, U, V)
    return out.reshape(batch)


# speedup vs baseline: 1.0013x; 1.0013x over previous
import functools

import jax
import jax.numpy as jnp
from jax import lax
from jax.experimental import pallas as pl
from jax.experimental.pallas import tpu as pltpu
from jax.experimental.pallas import tpu_sc as plsc

_NUM_WORKERS = 32
_RANK = 64
_LANES = 16
_CH = 128


def _make_sc_kernel(batch, nu_pairs, nv_pairs):
    bpw = batch // _NUM_WORKERS
    nch = bpw // _CH

    mesh = plsc.VectorSubcoreMesh(core_axis_name="c", subcore_axis_name="s")

    @functools.partial(
        pl.kernel,
        out_type=jax.ShapeDtypeStruct((batch,), jnp.float32),
        mesh=mesh,
        compiler_params=pltpu.CompilerParams(use_tc_tiling_on_sc=True),
        scratch_types=(
            [
                pltpu.VMEM((bpw,), jnp.int32),
                pltpu.VMEM((bpw,), jnp.int32),
            ]
            + [pltpu.VMEM((_CH,), jnp.int32) for _ in range(2 * (bpw // _CH))]
            + [
                pltpu.VMEM((2, _CH, 2 * _RANK), jnp.float32),
                pltpu.VMEM((2, _CH, 2 * _RANK), jnp.float32),
                pltpu.VMEM((bpw,), jnp.float32),
                pltpu.SemaphoreType.DMA,
                pltpu.SemaphoreType.DMA,
            ]
        ),
    )
    def sc_kernel(users_hbm, items_hbm, up_hbm, vp_hbm, out_hbm,
                  uraw, iraw, *rest):
        pu = rest[:nch]
        pi = rest[nch:2 * nch]
        ubuf, vbuf, outv, sem0, sem1 = rest[2 * nch:]
        sems = [sem0, sem1]

        cid = lax.axis_index("c")
        sid = lax.axis_index("s")
        wid = sid * 2 + cid
        base = wid * bpw

        pltpu.sync_copy(users_hbm.at[pl.ds(base, bpw)], uraw)
        pltpu.sync_copy(items_hbm.at[pl.ds(base, bpw)], iraw)

        for j in range(nch):
            for c in range(_CH // _LANES):
                off = j * _CH + c * _LANES
                pu[j][pl.ds(c * _LANES, _LANES)] = jnp.minimum(
                    lax.shift_right_logical(uraw[pl.ds(off, _LANES)], 1),
                    nu_pairs - 1)
                pi[j][pl.ds(c * _LANES, _LANES)] = jnp.minimum(
                    lax.shift_right_logical(iraw[pl.ds(off, _LANES)], 1),
                    nv_pairs - 1)

        def fire(j):
            b = j % 2
            pltpu.async_copy(up_hbm.at[pu[j]], ubuf.at[b], sems[b])
            pltpu.async_copy(vp_hbm.at[pi[j]], vbuf.at[b], sems[b])

        def wait(j):
            b = j % 2
            pltpu.make_async_copy(up_hbm.at[pu[j]], ubuf.at[b], sems[b]).wait()
            pltpu.make_async_copy(vp_hbm.at[pi[j]], vbuf.at[b], sems[b]).wait()

        perms = [jnp.arange(_LANES, dtype=jnp.int32) ^ d for d in (8, 4, 2, 1)]
        onehot = [lax.iota(jnp.int32, _LANES) == i for i in range(_LANES)]

        def lane_sum(x):
            for p in perms:
                x = x + x.at[p].get(mode="promise_in_bounds")
            return x

        fire(0)
        for j in range(nch):
            if j + 1 < nch:
                fire(j + 1)
            wait(j)
            b = j % 2

            def body(g, _, j=j, b=b):
                res = jnp.zeros((_LANES,), jnp.float32)
                e0 = j * _CH + g * _LANES
                upar = (uraw[pl.ds(e0, _LANES)] & 1) * _RANK
                ipar = (iraw[pl.ds(e0, _LANES)] & 1) * _RANK
                for i in range(_LANES):
                    r = g * _LANES + i
                    offu = upar[i]
                    offv = ipar[i]
                    acc = (ubuf[b, r, pl.ds(offu, _LANES)]
                           * vbuf[b, r, pl.ds(offv, _LANES)])
                    for c in range(1, _RANK // _LANES):
                        acc = acc + (
                            ubuf[b, r, pl.ds(offu + c * _LANES, _LANES)]
                            * vbuf[b, r, pl.ds(offv + c * _LANES, _LANES)])
                    res = jnp.where(onehot[i], lane_sum(acc), res)
                outv[pl.ds(j * _CH + g * _LANES, _LANES)] = res
                return 0

            lax.fori_loop(0, _CH // _LANES, body, 0)

        pltpu.sync_copy(outv, out_hbm.at[pl.ds(base, bpw)])

    return sc_kernel


def _pair(table):
    n = table.shape[0]
    return table[:n - n % 2].reshape(n // 2, 2 * table.shape[1])


def kernel(users, items, U, V):
    batch = users.shape[0]
    up, vp = _pair(U), _pair(V)
    out = _make_sc_kernel(batch, up.shape[0], vp.shape[0])(
        users.astype(jnp.int32), items.astype(jnp.int32), up, vp)
    return out
